# baseline (speedup 1.0000x reference)
import jax
import jax.numpy as jnp
from jax.experimental import pallas as pl
from jax.experimental.pallas import tpu as pltpu

HID = 64
LAT = 32
EDIM = 4
FIN = 26
FOUT = 20
N = 128
EID = 2 * HID + 1 + EDIM
BI = 32

_F32 = jnp.float32
_BF16 = jnp.bfloat16


def _silu(x):
    return x * jax.nn.sigmoid(x)


def _r(x):
    return x.astype(_BF16).astype(_F32)


def _vgae_kernel(feats_ref, coors_ref, edges_ref, eps_ref,
                 eW1_ref, eb1_ref, eW2_ref, eb2_ref,
                 nW1_ref, nb1_ref, nW2_ref, nb2_ref,
                 cW1_ref, cb1_ref, cW2_ref, misc_ref, lng_ref, lnb_ref,
                 embW_ref, embb_ref, fc1W_ref, fc1b_ref, fc2W_ref,
                 fc2b_ref, fc3W_ref, fc3b_ref,
                 xhat_ref, mu_ref, logvar_ref,
                 co_s, mi_s, dl_s, hb_s):
    f = feats_ref[0]
    h = jnp.dot(f.astype(_BF16), embW_ref[...],
                preferred_element_type=_F32) + embb_ref[...]
    co_s[...] = coors_ref[0]

    for l in range(3):
        hb_s[...] = h.astype(_BF16)

        def body(ib, carry):
            i0 = pl.multiple_of(ib * BI, BI)
            misc = misc_ref[l]
            c_b2 = misc[:, 0:1]
            cscale = misc[:, 1:2]
            co = co_s[...]
            ci = co_s[pl.ds(i0, BI), :]
            relc = ci[:, None, :] - co[None, :, :]
            rd = ((relc[..., 0:1] * relc[..., 0:1]
                   + relc[..., 1:2] * relc[..., 1:2])
                  + relc[..., 2:3] * relc[..., 2:3])
            hbi = hb_s[pl.ds(i0, BI), :]
            hbj = hb_s[...]
            ei = jnp.concatenate(
                [jnp.broadcast_to(hbi[:, None, :], (BI, N, HID)),
                 jnp.broadcast_to(hbj[None, :, :], (BI, N, HID)),
                 rd.astype(_BF16),
                 edges_ref[0, pl.ds(i0, BI), :, :]],
                axis=-1)
            E1 = jnp.dot(ei.reshape(BI * N, EID), eW1_ref[l],
                         preferred_element_type=_F32) + eb1_ref[l]
            h1b = _silu(E1).astype(_BF16)
            m2 = _silu(jnp.dot(h1b, eW2_ref[l],
                               preferred_element_type=_F32) + eb2_ref[l])
            s2b = _silu(jnp.dot(m2.astype(_BF16), cW1_ref[l],
                                preferred_element_type=_F32)
                        + cb1_ref[l]).astype(_BF16)
            cw = jnp.dot(s2b, cW2_ref[l],
                         preferred_element_type=_F32) + c_b2
            cw3 = _r(cw.reshape(BI, N, 1))
            norm = jnp.sqrt(rd)
            u = _r(relc / jnp.maximum(norm, 1e-8) * cscale[None])
            mi_s[pl.ds(i0, BI), :] = m2.reshape(BI, N, HID).sum(axis=1)
            dl_s[pl.ds(i0, BI), :] = (cw3 * u).sum(axis=1)
            return carry

        jax.lax.fori_loop(0, N // BI, body, 0)
        m_i = mi_s[...]
        co_s[...] = co_s[...] + dl_s[...]

        mean = h.mean(-1, keepdims=True)
        var = ((h - mean) ** 2).mean(-1, keepdims=True)
        normed = (h - mean) / jnp.sqrt(var + 1e-5) * lng_ref[l] + lnb_ref[l]
        ni = jnp.concatenate([normed, m_i], axis=-1)
        nh = _silu(jnp.dot(ni.astype(_BF16), nW1_ref[l],
                           preferred_element_type=_F32) + nb1_ref[l])
        h = jnp.dot(nh.astype(_BF16), nW2_ref[l],
                    preferred_element_type=_F32) + nb2_ref[l] + h

    hb = h.astype(_BF16)
    mu = jnp.dot(hb, fc1W_ref[...], preferred_element_type=_F32) \
        + fc1b_ref[...]
    logvar = jnp.dot(hb, fc2W_ref[...], preferred_element_type=_F32) \
        + fc2b_ref[...]
    std = jnp.exp(logvar * 0.5)
    z = mu + eps_ref[0] * std
    xhat_ref[0] = jnp.dot(z.astype(_BF16), fc3W_ref[...],
                          preferred_element_type=_F32) + fc3b_ref[...]
    mu_ref[0] = mu
    logvar_ref[0] = logvar


def kernel(feats, coors, edge_index, eps_noise, params):
    B = feats.shape[0]
    p = params
    layers = [p['l1'], p['l2'], p['l3']]

    def stk(name):
        return jnp.stack([l[name] for l in layers])

    def bf(a):
        return a.astype(_BF16)

    eW1 = bf(stk('e_W1'))
    eb1 = stk('e_b1')[:, None, :]
    eW2 = bf(stk('e_W2'))
    eb2 = stk('e_b2')[:, None, :]
    nW1 = bf(stk('n_W1'))
    nb1 = stk('n_b1')[:, None, :]
    nW2 = bf(stk('n_W2'))
    nb2 = stk('n_b2')[:, None, :]
    cW1 = bf(stk('c_W1'))
    cb1 = stk('c_b1')[:, None, :]
    cW2 = bf(stk('c_W2'))
    misc = jnp.stack([jnp.concatenate([l['c_b2'], l['coors_scale']])
                      for l in layers])[:, None, :]
    lng = stk('ln_g')[:, None, :]
    lnb = stk('ln_b')[:, None, :]

    edges_b = bf(edge_index)

    def full(a):
        r = a.ndim
        return pl.BlockSpec(a.shape, lambda b, _r=r: (0,) * _r)

    weights = [eW1, eb1, eW2, eb2, nW1, nb1, nW2, nb2,
               cW1, cb1, cW2, misc, lng, lnb,
               bf(p['emb_W']), p['emb_b'][None, :],
               bf(p['fc1_W']), p['fc1_b'][None, :],
               bf(p['fc2_W']), p['fc2_b'][None, :],
               bf(p['fc3_W']), p['fc3_b'][None, :]]

    in_specs = [
        pl.BlockSpec((1, N, FIN), lambda b: (b, 0, 0)),
        pl.BlockSpec((1, N, 3), lambda b: (b, 0, 0)),
        pl.BlockSpec((1, N, N, EDIM), lambda b: (b, 0, 0, 0)),
        pl.BlockSpec((1, N, LAT), lambda b: (b, 0, 0)),
    ] + [full(w) for w in weights]

    out_specs = [
        pl.BlockSpec((1, N, FOUT), lambda b: (b, 0, 0)),
        pl.BlockSpec((1, N, LAT), lambda b: (b, 0, 0)),
        pl.BlockSpec((1, N, LAT), lambda b: (b, 0, 0)),
    ]
    out_shape = [
        jax.ShapeDtypeStruct((B, N, FOUT), _F32),
        jax.ShapeDtypeStruct((B, N, LAT), _F32),
        jax.ShapeDtypeStruct((B, N, LAT), _F32),
    ]

    x_hat, mu, logvar = pl.pallas_call(
        _vgae_kernel,
        grid=(B,),
        in_specs=in_specs,
        out_specs=out_specs,
        out_shape=out_shape,
        scratch_shapes=[
            pltpu.VMEM((N, 3), _F32),
            pltpu.VMEM((N, HID), _F32),
            pltpu.VMEM((N, 3), _F32),
            pltpu.VMEM((N, HID), _BF16),
        ],
    )(feats, coors, edges_b, eps_noise, *weights)
    return (x_hat, mu, logvar)
